# baseline (speedup 1.0000x reference)
import functools

import jax
import jax.numpy as jnp
from jax import lax
from jax.experimental import pallas as pl
from jax.experimental.pallas import tpu as pltpu
from jax.experimental.pallas import tpu_sc as plsc

N = 10000
D = 128
E = 320000

NUM_CORES = 2
NUM_SUBCORES = 16
LANES = 16
EDGE_BLOCK = 128
NB = 80
PE = NUM_CORES * NUM_SUBCORES * NB * EDGE_BLOCK
N_PAD = 10240
ROWS_PER_TILE = N_PAD // NUM_SUBCORES


def _make_sc_aggregate(n_pad, nb, edge_block, d=D, interpret=False):
  rows_per_tile = n_pad // NUM_SUBCORES

  chunk = 8
  nchunks = nb // chunk

  def body(y_hbm, cols_hbm, rows_hbm, vals_hbm, zeros_hbm,
           out_hbm, cols_v, rows_v, vals_v, gath, acc, sem):
    c = lax.axis_index("c")
    s = lax.axis_index("s")

    pltpu.sync_copy(zeros_hbm, acc.at[pl.ds(s * rows_per_tile, rows_per_tile)])

    plsc.subcore_barrier()

    def chunk_body(g, carry):
        pltpu.sync_copy(cols_hbm.at[c, s, pl.ds(g * chunk, chunk)], cols_v)
        pltpu.sync_copy(rows_hbm.at[c, s, pl.ds(g * chunk, chunk)], rows_v)
        pltpu.sync_copy(
            vals_hbm.at[c, s, pl.ds(g * chunk * edge_block,
                                    chunk * edge_block)], vals_v)

        def block(k, kcarry):
            pltpu.async_copy(y_hbm.at[cols_v.at[k]], gath, sem).wait()

            base = k * edge_block
            for e in range(edge_block):
                fvec = jnp.full((LANES,), base + e, jnp.int32)
                val = plsc.load_gather(vals_v, [fvec])
                for j in range(d // LANES):
                    sl = pl.ds(j * LANES, LANES)
                    gath[e, sl] = gath[e, sl] * val

            pltpu.sync_copy(gath, acc.at[rows_v.at[k]], add=True)
            return kcarry

        lax.fori_loop(0, chunk, block, 0)
        return carry

    lax.fori_loop(0, nchunks, chunk_body, 0)

    plsc.subcore_barrier()
    sl = pl.ds(s * rows_per_tile, rows_per_tile)
    pltpu.sync_copy(acc.at[sl], out_hbm.at[c, sl])

  return pl.kernel(
      body,
      out_type=jax.ShapeDtypeStruct((NUM_CORES, n_pad, d), jnp.float32),
      mesh=plsc.VectorSubcoreMesh(
          core_axis_name="c", subcore_axis_name="s",
          num_cores=NUM_CORES, num_subcores=NUM_SUBCORES),
      compiler_params=pltpu.CompilerParams(needs_layout_passes=False),
      interpret=interpret,
      scratch_types=[
          pltpu.VMEM((8, edge_block), jnp.int32),
          pltpu.VMEM((8, edge_block), jnp.int32),
          pltpu.VMEM((8 * edge_block,), jnp.float32),
          pltpu.VMEM((edge_block, d), jnp.float32),
          pltpu.VMEM_SHARED((n_pad, d), jnp.float32),
          pltpu.SemaphoreType.DMA,
      ],
  )


_sc_aggregate = _make_sc_aggregate(N_PAD, NB, EDGE_BLOCK)


def _mm_body(x_ref, w_ref, y_ref):
    y_ref[...] = lax.dot_general(
        x_ref[...], w_ref[...], (((1,), (1,)), ((), ())),
        preferred_element_type=jnp.float32)


_mm = pl.pallas_call(
    _mm_body,
    out_shape=jax.ShapeDtypeStruct((N, D), jnp.float32),
)


def _normalize(h):
    norm = jnp.sqrt(jnp.sum(h * h, axis=-1, keepdims=True))
    return h / jnp.maximum(norm, 1e-12)


def _norm_mm_body(p_ref, w_ref, h_ref, y_ref):
    agg = p_ref[0, :N] + p_ref[1, :N]
    h_ref[...] = _normalize(agg)
    y_ref[...] = lax.dot_general(
        agg, w_ref[...], (((1,), (1,)), ((), ())),
        preferred_element_type=jnp.float32)


_norm_mm = pl.pallas_call(
    _norm_mm_body,
    out_shape=(jax.ShapeDtypeStruct((N, D), jnp.float32),
               jax.ShapeDtypeStruct((N, D), jnp.float32)),
)


def _final_body(e_ref, h1_ref, p_ref, o_ref):
    h2 = _normalize(p_ref[0, :N] + p_ref[1, :N])
    o_ref[...] = (e_ref[...] + h1_ref[...] + h2) * (1.0 / 3.0)


_final = pl.pallas_call(
    _final_body,
    out_shape=jax.ShapeDtypeStruct((N, D), jnp.float32),
)


def kernel(embedding, adj_row, adj_col, adj_values, W0, W1):
    pad = PE - E
    cols = jnp.pad(adj_col.astype(jnp.int32), (0, pad)).reshape(
        NUM_CORES, NUM_SUBCORES, NB, EDGE_BLOCK)
    rows = jnp.pad(adj_row.astype(jnp.int32), (0, pad)).reshape(
        NUM_CORES, NUM_SUBCORES, NB, EDGE_BLOCK)
    vals = jnp.pad(adj_values, (0, pad)).reshape(
        NUM_CORES, NUM_SUBCORES, NB * EDGE_BLOCK)
    zeros = jnp.zeros((ROWS_PER_TILE, D), jnp.float32)

    y0 = _mm(embedding, W0)
    p1 = _sc_aggregate(y0, cols, rows, vals, zeros)
    h1, y1 = _norm_mm(p1, W1)
    p2 = _sc_aggregate(y1, cols, rows, vals, zeros)
    return _final(embedding, h1, p2)


# speedup vs baseline: 1.1940x; 1.1940x over previous
import functools

import jax
import jax.numpy as jnp
from jax import lax
from jax.experimental import pallas as pl
from jax.experimental.pallas import tpu as pltpu
from jax.experimental.pallas import tpu_sc as plsc

N = 10000
D = 128
E = 320000

NUM_CORES = 2
NUM_SUBCORES = 16
LANES = 16
EDGE_BLOCK = 128
NB = 80
PE = NUM_CORES * NUM_SUBCORES * NB * EDGE_BLOCK
N_PAD = 10240
ROWS_PER_TILE = N_PAD // NUM_SUBCORES


def _make_sc_aggregate(n_pad, nb, edge_block, d=D, interpret=False):
  rows_per_tile = n_pad // NUM_SUBCORES

  chunk = 8
  nchunks = nb // chunk

  def body(y_hbm, cols_hbm, rows_hbm, vals_hbm, zeros_hbm,
           out_hbm, cols_v, rows_v, vals_v, gath, acc, sem):
    c = lax.axis_index("c")
    s = lax.axis_index("s")

    pltpu.sync_copy(zeros_hbm, acc.at[pl.ds(s * rows_per_tile, rows_per_tile)])

    plsc.subcore_barrier()

    def chunk_body(g, carry):
        pltpu.sync_copy(cols_hbm.at[c, s, pl.ds(g * chunk, chunk)], cols_v)
        pltpu.sync_copy(rows_hbm.at[c, s, pl.ds(g * chunk, chunk)], rows_v)
        pltpu.sync_copy(
            vals_hbm.at[c, s, pl.ds(g * chunk * edge_block,
                                    chunk * edge_block)], vals_v)

        def block(k, kcarry):
            pltpu.async_copy(y_hbm.at[cols_v.at[k]], gath, sem).wait()

            base = k * edge_block

            @plsc.parallel_loop(0, edge_block, unroll=8)
            def _scale(e):
                fvec = jnp.full((LANES,), base + e, jnp.int32)
                val = plsc.load_gather(vals_v, [fvec])
                for j in range(d // LANES):
                    sl = pl.ds(j * LANES, LANES)
                    gath[e, sl] = gath[e, sl] * val

            pltpu.sync_copy(gath, acc.at[rows_v.at[k]], add=True)
            return kcarry

        lax.fori_loop(0, chunk, block, 0)
        return carry

    lax.fori_loop(0, nchunks, chunk_body, 0)

    plsc.subcore_barrier()
    sl = pl.ds(s * rows_per_tile, rows_per_tile)
    pltpu.sync_copy(acc.at[sl], out_hbm.at[c, sl])

  return pl.kernel(
      body,
      out_type=jax.ShapeDtypeStruct((NUM_CORES, n_pad, d), jnp.float32),
      mesh=plsc.VectorSubcoreMesh(
          core_axis_name="c", subcore_axis_name="s",
          num_cores=NUM_CORES, num_subcores=NUM_SUBCORES),
      compiler_params=pltpu.CompilerParams(needs_layout_passes=False),
      interpret=interpret,
      scratch_types=[
          pltpu.VMEM((8, edge_block), jnp.int32),
          pltpu.VMEM((8, edge_block), jnp.int32),
          pltpu.VMEM((8 * edge_block,), jnp.float32),
          pltpu.VMEM((edge_block, d), jnp.float32),
          pltpu.VMEM_SHARED((n_pad, d), jnp.float32),
          pltpu.SemaphoreType.DMA,
      ],
  )


_sc_aggregate = _make_sc_aggregate(N_PAD, NB, EDGE_BLOCK)


def _mm_body(x_ref, w_ref, y_ref):
    y_ref[...] = lax.dot_general(
        x_ref[...], w_ref[...], (((1,), (1,)), ((), ())),
        preferred_element_type=jnp.float32)


_mm = pl.pallas_call(
    _mm_body,
    out_shape=jax.ShapeDtypeStruct((N, D), jnp.float32),
)


def _normalize(h):
    norm = jnp.sqrt(jnp.sum(h * h, axis=-1, keepdims=True))
    return h / jnp.maximum(norm, 1e-12)


def _norm_mm_body(p_ref, w_ref, h_ref, y_ref):
    agg = p_ref[0, :N] + p_ref[1, :N]
    h_ref[...] = _normalize(agg)
    y_ref[...] = lax.dot_general(
        agg, w_ref[...], (((1,), (1,)), ((), ())),
        preferred_element_type=jnp.float32)


_norm_mm = pl.pallas_call(
    _norm_mm_body,
    out_shape=(jax.ShapeDtypeStruct((N, D), jnp.float32),
               jax.ShapeDtypeStruct((N, D), jnp.float32)),
)


def _final_body(e_ref, h1_ref, p_ref, o_ref):
    h2 = _normalize(p_ref[0, :N] + p_ref[1, :N])
    o_ref[...] = (e_ref[...] + h1_ref[...] + h2) * (1.0 / 3.0)


_final = pl.pallas_call(
    _final_body,
    out_shape=jax.ShapeDtypeStruct((N, D), jnp.float32),
)


def kernel(embedding, adj_row, adj_col, adj_values, W0, W1):
    pad = PE - E
    cols = jnp.pad(adj_col.astype(jnp.int32), (0, pad)).reshape(
        NUM_CORES, NUM_SUBCORES, NB, EDGE_BLOCK)
    rows = jnp.pad(adj_row.astype(jnp.int32), (0, pad)).reshape(
        NUM_CORES, NUM_SUBCORES, NB, EDGE_BLOCK)
    vals = jnp.pad(adj_values, (0, pad)).reshape(
        NUM_CORES, NUM_SUBCORES, NB * EDGE_BLOCK)
    zeros = jnp.zeros((ROWS_PER_TILE, D), jnp.float32)

    y0 = _mm(embedding, W0)
    p1 = _sc_aggregate(y0, cols, rows, vals, zeros)
    h1, y1 = _norm_mm(p1, W1)
    p2 = _sc_aggregate(y1, cols, rows, vals, zeros)
    return _final(embedding, h1, p2)


# speedup vs baseline: 1.4014x; 1.1737x over previous
import functools

import jax
import jax.numpy as jnp
from jax import lax
from jax.experimental import pallas as pl
from jax.experimental.pallas import tpu as pltpu
from jax.experimental.pallas import tpu_sc as plsc

N = 10000
D = 128
E = 320000

NUM_CORES = 2
NUM_SUBCORES = 16
LANES = 16
EDGE_BLOCK = 128
NB = 80
PE = NUM_CORES * NUM_SUBCORES * NB * EDGE_BLOCK
N_PAD = 10240
ROWS_PER_TILE = N_PAD // NUM_SUBCORES


def _make_sc_aggregate(n_pad, nb, edge_block, d=D, interpret=False):
  rows_per_tile = n_pad // NUM_SUBCORES
  chunk = 16
  nchunks = nb // chunk

  def body(y_hbm, cols_hbm, rows_hbm, vals_hbm, zeros_hbm, out_hbm,
           cols_a, cols_b, rows_a, rows_b, vals_a, vals_b, gath_a, gath_b,
           acc, sem_g0, sem_g1, sem_s0, sem_s1, sem_i):
    c = lax.axis_index("c")
    s = lax.axis_index("s")
    cols_ab = (cols_a, cols_b)
    rows_ab = (rows_a, rows_b)
    vals_ab = (vals_a, vals_b)

    pltpu.sync_copy(zeros_hbm, acc.at[pl.ds(s * rows_per_tile, rows_per_tile)])

    pltpu.sync_copy(cols_hbm.at[c, s, pl.ds(0, chunk)], cols_a)
    pltpu.sync_copy(rows_hbm.at[c, s, pl.ds(0, chunk)], rows_a)
    pltpu.sync_copy(vals_hbm.at[c, s, pl.ds(0, chunk * edge_block)], vals_a)

    plsc.subcore_barrier()

    dummy_g = y_hbm.at[pl.ds(0, edge_block)]
    dummy_cols = cols_hbm.at[0, 0, pl.ds(0, chunk)]
    dummy_vals = vals_hbm.at[0, 0, pl.ds(0, chunk * edge_block)]

    def drain(sem, dummy_src, dst):
        pltpu.make_async_copy(dummy_src, dst, sem).wait()

    def make_scale(vals_v, gath, k):
        base = k * edge_block

        @plsc.parallel_loop(0, edge_block, unroll=8)
        def _scale(e):
            fvec = jnp.full((LANES,), base + e, jnp.int32)
            val = plsc.load_gather(vals_v, [fvec])
            for j in range(d // LANES):
                sl = pl.ds(j * LANES, LANES)
                gath[e, sl] = gath[e, sl] * val

    for g in range(nchunks):
        pc, pn = g % 2, (g + 1) % 2
        cols_c, rows_c, vals_c = cols_ab[pc], rows_ab[pc], vals_ab[pc]
        if g > 0:
            drain(sem_i, dummy_cols, cols_c)
            drain(sem_i, dummy_cols, rows_c)
            drain(sem_i, dummy_vals, vals_c)
        if g + 1 < nchunks:
            lo = (g + 1) * chunk
            pltpu.async_copy(cols_hbm.at[c, s, pl.ds(lo, chunk)],
                             cols_ab[pn], sem_i)
            pltpu.async_copy(rows_hbm.at[c, s, pl.ds(lo, chunk)],
                             rows_ab[pn], sem_i)
            pltpu.async_copy(vals_hbm.at[c, s, pl.ds(lo * edge_block,
                                                     chunk * edge_block)],
                             vals_ab[pn], sem_i)

        pltpu.async_copy(y_hbm.at[cols_c.at[0]], gath_a, sem_g0)
        pltpu.async_copy(y_hbm.at[cols_c.at[1]], gath_b, sem_g1)
        drain(sem_g0, dummy_g, gath_a)
        make_scale(vals_c, gath_a, 0)
        pltpu.async_copy(gath_a, acc.at[rows_c.at[0]], sem_s0, add=True)

        def pair(i, carry):
            k1 = 2 * i + 1
            drain(sem_s0, dummy_g, gath_a)
            pltpu.async_copy(y_hbm.at[cols_c.at[k1 + 1]], gath_a, sem_g0)
            drain(sem_g1, dummy_g, gath_b)
            make_scale(vals_c, gath_b, k1)
            pltpu.async_copy(gath_b, acc.at[rows_c.at[k1]], sem_s1, add=True)

            k2 = 2 * i + 2
            drain(sem_s1, dummy_g, gath_b)
            pltpu.async_copy(y_hbm.at[cols_c.at[k2 + 1]], gath_b, sem_g1)
            drain(sem_g0, dummy_g, gath_a)
            make_scale(vals_c, gath_a, k2)
            pltpu.async_copy(gath_a, acc.at[rows_c.at[k2]], sem_s0, add=True)
            return carry

        lax.fori_loop(0, (chunk - 2) // 2, pair, 0)

        drain(sem_s0, dummy_g, gath_a)
        drain(sem_g1, dummy_g, gath_b)
        make_scale(vals_c, gath_b, chunk - 1)
        pltpu.async_copy(gath_b, acc.at[rows_c.at[chunk - 1]], sem_s1,
                         add=True)
        drain(sem_s1, dummy_g, gath_b)

    plsc.subcore_barrier()
    sl = pl.ds(s * rows_per_tile, rows_per_tile)
    pltpu.sync_copy(acc.at[sl], out_hbm.at[c, sl])

  return pl.kernel(
      body,
      out_type=jax.ShapeDtypeStruct((NUM_CORES, n_pad, d), jnp.float32),
      mesh=plsc.VectorSubcoreMesh(
          core_axis_name="c", subcore_axis_name="s",
          num_cores=NUM_CORES, num_subcores=NUM_SUBCORES),
      compiler_params=pltpu.CompilerParams(needs_layout_passes=False),
      interpret=interpret,
      scratch_types=[
          pltpu.VMEM((chunk, edge_block), jnp.int32),
          pltpu.VMEM((chunk, edge_block), jnp.int32),
          pltpu.VMEM((chunk, edge_block), jnp.int32),
          pltpu.VMEM((chunk, edge_block), jnp.int32),
          pltpu.VMEM((chunk * edge_block,), jnp.float32),
          pltpu.VMEM((chunk * edge_block,), jnp.float32),
          pltpu.VMEM((edge_block, d), jnp.float32),
          pltpu.VMEM((edge_block, d), jnp.float32),
          pltpu.VMEM_SHARED((n_pad, d), jnp.float32),
          pltpu.SemaphoreType.DMA,
          pltpu.SemaphoreType.DMA,
          pltpu.SemaphoreType.DMA,
          pltpu.SemaphoreType.DMA,
          pltpu.SemaphoreType.DMA,
      ],
  )


_sc_aggregate = _make_sc_aggregate(N_PAD, NB, EDGE_BLOCK)


def _mm_body(x_ref, w_ref, y_ref):
    y_ref[...] = lax.dot_general(
        x_ref[...], w_ref[...], (((1,), (1,)), ((), ())),
        preferred_element_type=jnp.float32)


_mm = pl.pallas_call(
    _mm_body,
    out_shape=jax.ShapeDtypeStruct((N, D), jnp.float32),
)


def _normalize(h):
    norm = jnp.sqrt(jnp.sum(h * h, axis=-1, keepdims=True))
    return h / jnp.maximum(norm, 1e-12)


def _norm_mm_body(p_ref, w_ref, h_ref, y_ref):
    agg = p_ref[0, :N] + p_ref[1, :N]
    h_ref[...] = _normalize(agg)
    y_ref[...] = lax.dot_general(
        agg, w_ref[...], (((1,), (1,)), ((), ())),
        preferred_element_type=jnp.float32)


_norm_mm = pl.pallas_call(
    _norm_mm_body,
    out_shape=(jax.ShapeDtypeStruct((N, D), jnp.float32),
               jax.ShapeDtypeStruct((N, D), jnp.float32)),
)


def _final_body(e_ref, h1_ref, p_ref, o_ref):
    h2 = _normalize(p_ref[0, :N] + p_ref[1, :N])
    o_ref[...] = (e_ref[...] + h1_ref[...] + h2) * (1.0 / 3.0)


_final = pl.pallas_call(
    _final_body,
    out_shape=jax.ShapeDtypeStruct((N, D), jnp.float32),
)


def kernel(embedding, adj_row, adj_col, adj_values, W0, W1):
    pad = PE - E
    cols = jnp.pad(adj_col.astype(jnp.int32), (0, pad)).reshape(
        NUM_CORES, NUM_SUBCORES, NB, EDGE_BLOCK)
    rows = jnp.pad(adj_row.astype(jnp.int32), (0, pad)).reshape(
        NUM_CORES, NUM_SUBCORES, NB, EDGE_BLOCK)
    vals = jnp.pad(adj_values, (0, pad)).reshape(
        NUM_CORES, NUM_SUBCORES, NB * EDGE_BLOCK)
    zeros = jnp.zeros((ROWS_PER_TILE, D), jnp.float32)

    y0 = _mm(embedding, W0)
    p1 = _sc_aggregate(y0, cols, rows, vals, zeros)
    h1, y1 = _norm_mm(p1, W1)
    p2 = _sc_aggregate(y1, cols, rows, vals, zeros)
    return _final(embedding, h1, p2)


# speedup vs baseline: 1.4025x; 1.0008x over previous
import functools

import jax
import jax.numpy as jnp
from jax import lax
from jax.experimental import pallas as pl
from jax.experimental.pallas import tpu as pltpu
from jax.experimental.pallas import tpu_sc as plsc

N = 10000
D = 128
E = 320000

NUM_CORES = 2
NUM_SUBCORES = 16
LANES = 16
EDGE_BLOCK = 128
NB = 80
PE = NUM_CORES * NUM_SUBCORES * NB * EDGE_BLOCK
N_PAD = 10240
ROWS_PER_TILE = N_PAD // NUM_SUBCORES


def _make_sc_aggregate(n_pad, nb, edge_block, d=D, interpret=False):
  rows_per_tile = n_pad // NUM_SUBCORES
  chunk = 16
  nchunks = nb // chunk

  def body(y_hbm, cols_hbm, rows_hbm, vals_hbm, zeros_hbm, out_hbm,
           cols_a, cols_b, rows_a, rows_b, vals_a, vals_b, gath_a, gath_b,
           acc, sem_g0, sem_g1, sem_s0, sem_s1, sem_i):
    c = lax.axis_index("c")
    s = lax.axis_index("s")
    cols_ab = (cols_a, cols_b)
    rows_ab = (rows_a, rows_b)
    vals_ab = (vals_a, vals_b)

    pltpu.sync_copy(zeros_hbm, acc.at[pl.ds(s * rows_per_tile, rows_per_tile)])

    pltpu.sync_copy(cols_hbm.at[c, s, pl.ds(0, chunk)], cols_a)
    pltpu.sync_copy(rows_hbm.at[c, s, pl.ds(0, chunk)], rows_a)
    pltpu.sync_copy(vals_hbm.at[c, s, pl.ds(0, chunk * edge_block)], vals_a)

    plsc.subcore_barrier()

    dummy_g = y_hbm.at[pl.ds(0, edge_block)]
    dummy_cols = cols_hbm.at[0, 0, pl.ds(0, chunk)]
    dummy_vals = vals_hbm.at[0, 0, pl.ds(0, chunk * edge_block)]

    def drain(sem, dummy_src, dst):
        pltpu.make_async_copy(dummy_src, dst, sem).wait()

    def _gather4(y, cols, k, buf, sem):
        q = edge_block // 4
        for h in range(4):
            pltpu.async_copy(y.at[cols.at[k, pl.ds(h * q, q)]],
                             buf.at[pl.ds(h * q, q)], sem)

    def make_scale(vals_v, gath, k):
        base = k * edge_block

        @plsc.parallel_loop(0, edge_block, unroll=8)
        def _scale(e):
            fvec = jnp.full((LANES,), base + e, jnp.int32)
            val = plsc.load_gather(vals_v, [fvec])
            for j in range(d // LANES):
                sl = pl.ds(j * LANES, LANES)
                gath[e, sl] = gath[e, sl] * val

    for g in range(nchunks):
        pc, pn = g % 2, (g + 1) % 2
        cols_c, rows_c, vals_c = cols_ab[pc], rows_ab[pc], vals_ab[pc]
        if g > 0:
            drain(sem_i, dummy_cols, cols_c)
            drain(sem_i, dummy_cols, rows_c)
            drain(sem_i, dummy_vals, vals_c)
        if g + 1 < nchunks:
            lo = (g + 1) * chunk
            pltpu.async_copy(cols_hbm.at[c, s, pl.ds(lo, chunk)],
                             cols_ab[pn], sem_i)
            pltpu.async_copy(rows_hbm.at[c, s, pl.ds(lo, chunk)],
                             rows_ab[pn], sem_i)
            pltpu.async_copy(vals_hbm.at[c, s, pl.ds(lo * edge_block,
                                                     chunk * edge_block)],
                             vals_ab[pn], sem_i)

        _gather4(y_hbm, cols_c, 0, gath_a, sem_g0)
        _gather4(y_hbm, cols_c, 1, gath_b, sem_g1)
        drain(sem_g0, dummy_g, gath_a)
        make_scale(vals_c, gath_a, 0)
        pltpu.async_copy(gath_a, acc.at[rows_c.at[0]], sem_s0, add=True)

        def pair(i, carry):
            k1 = 2 * i + 1
            drain(sem_s0, dummy_g, gath_a)
            _gather4(y_hbm, cols_c, k1 + 1, gath_a, sem_g0)
            drain(sem_g1, dummy_g, gath_b)
            make_scale(vals_c, gath_b, k1)
            pltpu.async_copy(gath_b, acc.at[rows_c.at[k1]], sem_s1, add=True)

            k2 = 2 * i + 2
            drain(sem_s1, dummy_g, gath_b)
            _gather4(y_hbm, cols_c, k2 + 1, gath_b, sem_g1)
            drain(sem_g0, dummy_g, gath_a)
            make_scale(vals_c, gath_a, k2)
            pltpu.async_copy(gath_a, acc.at[rows_c.at[k2]], sem_s0, add=True)
            return carry

        lax.fori_loop(0, (chunk - 2) // 2, pair, 0)

        drain(sem_s0, dummy_g, gath_a)
        drain(sem_g1, dummy_g, gath_b)
        make_scale(vals_c, gath_b, chunk - 1)
        pltpu.async_copy(gath_b, acc.at[rows_c.at[chunk - 1]], sem_s1,
                         add=True)
        drain(sem_s1, dummy_g, gath_b)

    plsc.subcore_barrier()
    sl = pl.ds(s * rows_per_tile, rows_per_tile)
    pltpu.sync_copy(acc.at[sl], out_hbm.at[c, sl])

  return pl.kernel(
      body,
      out_type=jax.ShapeDtypeStruct((NUM_CORES, n_pad, d), jnp.float32),
      mesh=plsc.VectorSubcoreMesh(
          core_axis_name="c", subcore_axis_name="s",
          num_cores=NUM_CORES, num_subcores=NUM_SUBCORES),
      compiler_params=pltpu.CompilerParams(needs_layout_passes=False),
      interpret=interpret,
      scratch_types=[
          pltpu.VMEM((chunk, edge_block), jnp.int32),
          pltpu.VMEM((chunk, edge_block), jnp.int32),
          pltpu.VMEM((chunk, edge_block), jnp.int32),
          pltpu.VMEM((chunk, edge_block), jnp.int32),
          pltpu.VMEM((chunk * edge_block,), jnp.float32),
          pltpu.VMEM((chunk * edge_block,), jnp.float32),
          pltpu.VMEM((edge_block, d), jnp.float32),
          pltpu.VMEM((edge_block, d), jnp.float32),
          pltpu.VMEM_SHARED((n_pad, d), jnp.float32),
          pltpu.SemaphoreType.DMA,
          pltpu.SemaphoreType.DMA,
          pltpu.SemaphoreType.DMA,
          pltpu.SemaphoreType.DMA,
          pltpu.SemaphoreType.DMA,
      ],
  )


_sc_aggregate = _make_sc_aggregate(N_PAD, NB, EDGE_BLOCK)


def _mm_body(x_ref, w_ref, y_ref):
    y_ref[...] = lax.dot_general(
        x_ref[...], w_ref[...], (((1,), (1,)), ((), ())),
        preferred_element_type=jnp.float32)


_mm = pl.pallas_call(
    _mm_body,
    out_shape=jax.ShapeDtypeStruct((N, D), jnp.float32),
)


def _normalize(h):
    norm = jnp.sqrt(jnp.sum(h * h, axis=-1, keepdims=True))
    return h / jnp.maximum(norm, 1e-12)


def _norm_mm_body(p_ref, w_ref, h_ref, y_ref):
    agg = p_ref[0, :N] + p_ref[1, :N]
    h_ref[...] = _normalize(agg)
    y_ref[...] = lax.dot_general(
        agg, w_ref[...], (((1,), (1,)), ((), ())),
        preferred_element_type=jnp.float32)


_norm_mm = pl.pallas_call(
    _norm_mm_body,
    out_shape=(jax.ShapeDtypeStruct((N, D), jnp.float32),
               jax.ShapeDtypeStruct((N, D), jnp.float32)),
)


def _final_body(e_ref, h1_ref, p_ref, o_ref):
    h2 = _normalize(p_ref[0, :N] + p_ref[1, :N])
    o_ref[...] = (e_ref[...] + h1_ref[...] + h2) * (1.0 / 3.0)


_final = pl.pallas_call(
    _final_body,
    out_shape=jax.ShapeDtypeStruct((N, D), jnp.float32),
)


def kernel(embedding, adj_row, adj_col, adj_values, W0, W1):
    pad = PE - E
    cols = jnp.pad(adj_col.astype(jnp.int32), (0, pad)).reshape(
        NUM_CORES, NUM_SUBCORES, NB, EDGE_BLOCK)
    rows = jnp.pad(adj_row.astype(jnp.int32), (0, pad)).reshape(
        NUM_CORES, NUM_SUBCORES, NB, EDGE_BLOCK)
    vals = jnp.pad(adj_values, (0, pad)).reshape(
        NUM_CORES, NUM_SUBCORES, NB * EDGE_BLOCK)
    zeros = jnp.zeros((ROWS_PER_TILE, D), jnp.float32)

    y0 = _mm(embedding, W0)
    p1 = _sc_aggregate(y0, cols, rows, vals, zeros)
    h1, y1 = _norm_mm(p1, W1)
    p2 = _sc_aggregate(y1, cols, rows, vals, zeros)
    return _final(embedding, h1, p2)


# speedup vs baseline: 1.8997x; 1.3545x over previous
import functools

import jax
import jax.numpy as jnp
from jax import lax
from jax.experimental import pallas as pl
from jax.experimental.pallas import tpu as pltpu
from jax.experimental.pallas import tpu_sc as plsc

N = 10000
D = 128
E = 320000

NUM_CORES = 2
NUM_SUBCORES = 16
LANES = 16
EDGE_BLOCK = 128
NB = 80
PE = NUM_CORES * NUM_SUBCORES * NB * EDGE_BLOCK
N_PAD = 10240
ROWS_PER_TILE = N_PAD // NUM_SUBCORES


def _make_sc_aggregate(n_pad, nb, edge_block, d=D, interpret=False):
  rows_per_tile = n_pad // NUM_SUBCORES
  dh = d // NUM_CORES
  chunk = 16
  nchunks = nb // chunk

  def body(y_hbm, cols_hbm, rows_hbm, vals_hbm, zeros_hbm, out_hbm,
           cols_a, cols_b, rows_a, rows_b, vals_a, vals_b, gath_a, gath_b,
           ystore, acc, sem_g0, sem_g1, sem_s0, sem_s1, sem_i):
    c = lax.axis_index("c")
    s = lax.axis_index("s")
    cols_ab = (cols_a, cols_b)
    rows_ab = (rows_a, rows_b)
    vals_ab = (vals_a, vals_b)

    rsl = pl.ds(s * rows_per_tile, rows_per_tile)
    pltpu.sync_copy(zeros_hbm, acc.at[rsl])

    pltpu.sync_copy(cols_hbm.at[s, pl.ds(0, chunk)], cols_a)
    pltpu.sync_copy(rows_hbm.at[s, pl.ds(0, chunk)], rows_a)
    pltpu.sync_copy(vals_hbm.at[s, pl.ds(0, chunk * edge_block)], vals_a)

    plsc.subcore_barrier()

    dummy_g = y_hbm.at[0, pl.ds(0, edge_block)]
    dummy_cols = cols_hbm.at[0, pl.ds(0, chunk)]
    dummy_vals = vals_hbm.at[0, pl.ds(0, chunk * edge_block)]

    def drain(sem, dummy_src, dst):
        pltpu.make_async_copy(dummy_src, dst, sem).wait()

    def make_scale(vals_v, gath, k):
        base = k * edge_block

        @plsc.parallel_loop(0, edge_block, unroll=8)
        def _scale(e):
            fvec = jnp.full((LANES,), base + e, jnp.int32)
            val = plsc.load_gather(vals_v, [fvec])
            for j in range(dh // LANES):
                sl = pl.ds(j * LANES, LANES)
                gath[e, sl] = gath[e, sl] * val

    for g in range(nchunks):
        pc, pn = g % 2, (g + 1) % 2
        cols_c, rows_c, vals_c = cols_ab[pc], rows_ab[pc], vals_ab[pc]
        if g > 0:
            drain(sem_i, dummy_cols, cols_c)
            drain(sem_i, dummy_cols, rows_c)
            drain(sem_i, dummy_vals, vals_c)
        if g + 1 < nchunks:
            lo = (g + 1) * chunk
            pltpu.async_copy(cols_hbm.at[s, pl.ds(lo, chunk)],
                             cols_ab[pn], sem_i)
            pltpu.async_copy(rows_hbm.at[s, pl.ds(lo, chunk)],
                             rows_ab[pn], sem_i)
            pltpu.async_copy(vals_hbm.at[s, pl.ds(lo * edge_block,
                                                  chunk * edge_block)],
                             vals_ab[pn], sem_i)

        pltpu.async_copy(y_hbm.at[c].at[cols_c.at[0]], gath_a, sem_g0)
        pltpu.async_copy(y_hbm.at[c].at[cols_c.at[1]], gath_b, sem_g1)
        drain(sem_g0, dummy_g, gath_a)
        make_scale(vals_c, gath_a, 0)
        pltpu.async_copy(gath_a, acc.at[rows_c.at[0]], sem_s0, add=True)

        def pair(i, carry):
            k1 = 2 * i + 1
            drain(sem_s0, dummy_g, gath_a)
            pltpu.async_copy(y_hbm.at[c].at[cols_c.at[k1 + 1]], gath_a, sem_g0)
            drain(sem_g1, dummy_g, gath_b)
            make_scale(vals_c, gath_b, k1)
            pltpu.async_copy(gath_b, acc.at[rows_c.at[k1]], sem_s1, add=True)

            k2 = 2 * i + 2
            drain(sem_s1, dummy_g, gath_b)
            pltpu.async_copy(y_hbm.at[c].at[cols_c.at[k2 + 1]], gath_b, sem_g1)
            drain(sem_g0, dummy_g, gath_a)
            make_scale(vals_c, gath_a, k2)
            pltpu.async_copy(gath_a, acc.at[rows_c.at[k2]], sem_s0, add=True)
            return carry

        lax.fori_loop(0, (chunk - 2) // 2, pair, 0)

        drain(sem_s0, dummy_g, gath_a)
        drain(sem_g1, dummy_g, gath_b)
        make_scale(vals_c, gath_b, chunk - 1)
        pltpu.async_copy(gath_b, acc.at[rows_c.at[chunk - 1]], sem_s1,
                         add=True)
        drain(sem_s1, dummy_g, gath_b)

    plsc.subcore_barrier()
    pltpu.sync_copy(acc.at[rsl], out_hbm.at[c, rsl])

  return pl.kernel(
      body,
      out_type=jax.ShapeDtypeStruct((NUM_CORES, n_pad, d // NUM_CORES),
                                    jnp.float32),
      mesh=plsc.VectorSubcoreMesh(
          core_axis_name="c", subcore_axis_name="s",
          num_cores=NUM_CORES, num_subcores=NUM_SUBCORES),
      compiler_params=pltpu.CompilerParams(needs_layout_passes=False,
                                           use_tc_tiling_on_sc=False),
      interpret=interpret,
      scratch_types=[
          pltpu.VMEM((chunk, edge_block), jnp.int32),
          pltpu.VMEM((chunk, edge_block), jnp.int32),
          pltpu.VMEM((chunk, edge_block), jnp.int32),
          pltpu.VMEM((chunk, edge_block), jnp.int32),
          pltpu.VMEM((chunk * edge_block,), jnp.float32),
          pltpu.VMEM((chunk * edge_block,), jnp.float32),
          pltpu.VMEM((edge_block, D // NUM_CORES), jnp.float32),
          pltpu.VMEM((edge_block, D // NUM_CORES), jnp.float32),
          pltpu.VMEM_SHARED((n_pad, D // NUM_CORES), jnp.float32),
          pltpu.VMEM_SHARED((n_pad, D // NUM_CORES), jnp.float32),
          pltpu.SemaphoreType.DMA,
          pltpu.SemaphoreType.DMA,
          pltpu.SemaphoreType.DMA,
          pltpu.SemaphoreType.DMA,
          pltpu.SemaphoreType.DMA,
      ],
  )


NB2 = PE // (NUM_SUBCORES * EDGE_BLOCK)
_sc_aggregate = _make_sc_aggregate(N_PAD, NB2, EDGE_BLOCK)


def _mm_body(x_ref, w_ref, y_ref):
    y_ref[...] = lax.dot_general(
        x_ref[...], w_ref[...], (((1,), (1,)), ((), ())),
        preferred_element_type=jnp.float32)


_mm = pl.pallas_call(
    _mm_body,
    out_shape=jax.ShapeDtypeStruct((N, D), jnp.float32),
)


def _normalize(h):
    norm = jnp.sqrt(jnp.sum(h * h, axis=-1, keepdims=True))
    return h / jnp.maximum(norm, 1e-12)


def _agg_of(p_ref):
    return jnp.concatenate([p_ref[0, :N], p_ref[1, :N]], axis=-1)


def _norm_mm_body(p_ref, w_ref, h_ref, y_ref):
    agg = _agg_of(p_ref)
    h_ref[...] = _normalize(agg)
    y_ref[...] = lax.dot_general(
        agg, w_ref[...], (((1,), (1,)), ((), ())),
        preferred_element_type=jnp.float32)


_norm_mm = pl.pallas_call(
    _norm_mm_body,
    out_shape=(jax.ShapeDtypeStruct((N, D), jnp.float32),
               jax.ShapeDtypeStruct((N, D), jnp.float32)),
)


def _final_body(e_ref, h1_ref, p_ref, o_ref):
    h2 = _normalize(_agg_of(p_ref))
    o_ref[...] = (e_ref[...] + h1_ref[...] + h2) * (1.0 / 3.0)


_final = pl.pallas_call(
    _final_body,
    out_shape=jax.ShapeDtypeStruct((N, D), jnp.float32),
)


def kernel(embedding, adj_row, adj_col, adj_values, W0, W1):
    pad = PE - E
    cols = jnp.pad(adj_col.astype(jnp.int32), (0, pad)).reshape(
        NUM_SUBCORES, NB2, EDGE_BLOCK)
    rows = jnp.pad(adj_row.astype(jnp.int32), (0, pad)).reshape(
        NUM_SUBCORES, NB2, EDGE_BLOCK)
    vals = jnp.pad(adj_values, (0, pad)).reshape(
        NUM_SUBCORES, NB2 * EDGE_BLOCK)
    zeros = jnp.zeros((N_PAD // NUM_SUBCORES, D // NUM_CORES), jnp.float32)

    def split_cols(y):
        y = jnp.pad(y, ((0, N_PAD - N), (0, 0)))
        return jnp.stack([y[:, :D // 2], y[:, D // 2:]])

    y0 = _mm(embedding, W0)
    p1 = _sc_aggregate(split_cols(y0), cols, rows, vals, zeros)
    h1, y1 = _norm_mm(p1, W1)
    p2 = _sc_aggregate(split_cols(y1), cols, rows, vals, zeros)
    return _final(embedding, h1, p2)


# speedup vs baseline: 3.0874x; 1.6252x over previous
import functools

import jax
import jax.numpy as jnp
from jax import lax
from jax.experimental import pallas as pl
from jax.experimental.pallas import tpu as pltpu
from jax.experimental.pallas import tpu_sc as plsc

N = 10000
D = 128
E = 320000

NUM_CORES = 2
NUM_SUBCORES = 16
LANES = 16
EDGE_BLOCK = 128
NB = 80
PE = NUM_CORES * NUM_SUBCORES * NB * EDGE_BLOCK
N_PAD = 10240
ROWS_PER_TILE = N_PAD // NUM_SUBCORES


def _make_sc_aggregate(n_pad, nb, edge_block, d=D, interpret=False):
  rows_per_tile = n_pad // NUM_SUBCORES
  dh = d // NUM_CORES
  chunk = 16
  nchunks = nb // chunk

  def body(y_hbm, cols_hbm, rows_hbm, vals_hbm, zeros_hbm, out_hbm,
           cols_a, cols_b, rows_a, rows_b, vals_a, vals_b, gath_a, gath_b,
           ystore, acc, sem_g0, sem_g1, sem_s0, sem_s1, sem_i):
    c = lax.axis_index("c")
    s = lax.axis_index("s")
    cols_ab = (cols_a, cols_b)
    rows_ab = (rows_a, rows_b)
    vals_ab = (vals_a, vals_b)

    rsl = pl.ds(s * rows_per_tile, rows_per_tile)
    pltpu.sync_copy(y_hbm.at[c, rsl], ystore.at[rsl])
    pltpu.sync_copy(zeros_hbm, acc.at[rsl])

    pltpu.sync_copy(cols_hbm.at[s, pl.ds(0, chunk)], cols_a)
    pltpu.sync_copy(rows_hbm.at[s, pl.ds(0, chunk)], rows_a)
    pltpu.sync_copy(vals_hbm.at[s, pl.ds(0, chunk * edge_block)], vals_a)

    plsc.subcore_barrier()

    dummy_g = y_hbm.at[0, pl.ds(0, edge_block)]
    dummy_cols = cols_hbm.at[0, pl.ds(0, chunk)]
    dummy_vals = vals_hbm.at[0, pl.ds(0, chunk * edge_block)]

    def drain(sem, dummy_src, dst):
        pltpu.make_async_copy(dummy_src, dst, sem).wait()

    def make_scale(vals_v, gath, k):
        base = k * edge_block

        @plsc.parallel_loop(0, edge_block, unroll=8)
        def _scale(e):
            fvec = jnp.full((LANES,), base + e, jnp.int32)
            val = plsc.load_gather(vals_v, [fvec])
            for j in range(dh // LANES):
                sl = pl.ds(j * LANES, LANES)
                gath[e, sl] = gath[e, sl] * val

    for g in range(nchunks):
        pc, pn = g % 2, (g + 1) % 2
        cols_c, rows_c, vals_c = cols_ab[pc], rows_ab[pc], vals_ab[pc]
        if g > 0:
            drain(sem_i, dummy_cols, cols_c)
            drain(sem_i, dummy_cols, rows_c)
            drain(sem_i, dummy_vals, vals_c)
        if g + 1 < nchunks:
            lo = (g + 1) * chunk
            pltpu.async_copy(cols_hbm.at[s, pl.ds(lo, chunk)],
                             cols_ab[pn], sem_i)
            pltpu.async_copy(rows_hbm.at[s, pl.ds(lo, chunk)],
                             rows_ab[pn], sem_i)
            pltpu.async_copy(vals_hbm.at[s, pl.ds(lo * edge_block,
                                                  chunk * edge_block)],
                             vals_ab[pn], sem_i)

        pltpu.async_copy(ystore.at[cols_c.at[0]], gath_a, sem_g0)
        pltpu.async_copy(ystore.at[cols_c.at[1]], gath_b, sem_g1)
        drain(sem_g0, dummy_g, gath_a)
        make_scale(vals_c, gath_a, 0)
        pltpu.async_copy(gath_a, acc.at[rows_c.at[0]], sem_s0, add=True)

        def pair(i, carry):
            k1 = 2 * i + 1
            drain(sem_s0, dummy_g, gath_a)
            pltpu.async_copy(ystore.at[cols_c.at[k1 + 1]], gath_a, sem_g0)
            drain(sem_g1, dummy_g, gath_b)
            make_scale(vals_c, gath_b, k1)
            pltpu.async_copy(gath_b, acc.at[rows_c.at[k1]], sem_s1, add=True)

            k2 = 2 * i + 2
            drain(sem_s1, dummy_g, gath_b)
            pltpu.async_copy(ystore.at[cols_c.at[k2 + 1]], gath_b, sem_g1)
            drain(sem_g0, dummy_g, gath_a)
            make_scale(vals_c, gath_a, k2)
            pltpu.async_copy(gath_a, acc.at[rows_c.at[k2]], sem_s0, add=True)
            return carry

        lax.fori_loop(0, (chunk - 2) // 2, pair, 0)

        drain(sem_s0, dummy_g, gath_a)
        drain(sem_g1, dummy_g, gath_b)
        make_scale(vals_c, gath_b, chunk - 1)
        pltpu.async_copy(gath_b, acc.at[rows_c.at[chunk - 1]], sem_s1,
                         add=True)
        drain(sem_s1, dummy_g, gath_b)

    plsc.subcore_barrier()
    pltpu.sync_copy(acc.at[rsl], out_hbm.at[c, rsl])

  return pl.kernel(
      body,
      out_type=jax.ShapeDtypeStruct((NUM_CORES, n_pad, d // NUM_CORES),
                                    jnp.float32),
      mesh=plsc.VectorSubcoreMesh(
          core_axis_name="c", subcore_axis_name="s",
          num_cores=NUM_CORES, num_subcores=NUM_SUBCORES),
      compiler_params=pltpu.CompilerParams(needs_layout_passes=False,
                                           use_tc_tiling_on_sc=False),
      interpret=interpret,
      scratch_types=[
          pltpu.VMEM((chunk, edge_block), jnp.int32),
          pltpu.VMEM((chunk, edge_block), jnp.int32),
          pltpu.VMEM((chunk, edge_block), jnp.int32),
          pltpu.VMEM((chunk, edge_block), jnp.int32),
          pltpu.VMEM((chunk * edge_block,), jnp.float32),
          pltpu.VMEM((chunk * edge_block,), jnp.float32),
          pltpu.VMEM((edge_block, D // NUM_CORES), jnp.float32),
          pltpu.VMEM((edge_block, D // NUM_CORES), jnp.float32),
          pltpu.VMEM_SHARED((n_pad, D // NUM_CORES), jnp.float32),
          pltpu.VMEM_SHARED((n_pad, D // NUM_CORES), jnp.float32),
          pltpu.SemaphoreType.DMA,
          pltpu.SemaphoreType.DMA,
          pltpu.SemaphoreType.DMA,
          pltpu.SemaphoreType.DMA,
          pltpu.SemaphoreType.DMA,
      ],
  )


NB2 = PE // (NUM_SUBCORES * EDGE_BLOCK)
_sc_aggregate = _make_sc_aggregate(N_PAD, NB2, EDGE_BLOCK)


def _mm_body(x_ref, w_ref, y_ref):
    y_ref[...] = lax.dot_general(
        x_ref[...], w_ref[...], (((1,), (1,)), ((), ())),
        preferred_element_type=jnp.float32)


_mm = pl.pallas_call(
    _mm_body,
    out_shape=jax.ShapeDtypeStruct((N, D), jnp.float32),
)


def _normalize(h):
    norm = jnp.sqrt(jnp.sum(h * h, axis=-1, keepdims=True))
    return h / jnp.maximum(norm, 1e-12)


def _agg_of(p_ref):
    return jnp.concatenate([p_ref[0, :N], p_ref[1, :N]], axis=-1)


def _norm_mm_body(p_ref, w_ref, h_ref, y_ref):
    agg = _agg_of(p_ref)
    h_ref[...] = _normalize(agg)
    y_ref[...] = lax.dot_general(
        agg, w_ref[...], (((1,), (1,)), ((), ())),
        preferred_element_type=jnp.float32)


_norm_mm = pl.pallas_call(
    _norm_mm_body,
    out_shape=(jax.ShapeDtypeStruct((N, D), jnp.float32),
               jax.ShapeDtypeStruct((N, D), jnp.float32)),
)


def _final_body(e_ref, h1_ref, p_ref, o_ref):
    h2 = _normalize(_agg_of(p_ref))
    o_ref[...] = (e_ref[...] + h1_ref[...] + h2) * (1.0 / 3.0)


_final = pl.pallas_call(
    _final_body,
    out_shape=jax.ShapeDtypeStruct((N, D), jnp.float32),
)


def kernel(embedding, adj_row, adj_col, adj_values, W0, W1):
    pad = PE - E
    cols = jnp.pad(adj_col.astype(jnp.int32), (0, pad)).reshape(
        NUM_SUBCORES, NB2, EDGE_BLOCK)
    rows = jnp.pad(adj_row.astype(jnp.int32), (0, pad)).reshape(
        NUM_SUBCORES, NB2, EDGE_BLOCK)
    vals = jnp.pad(adj_values, (0, pad)).reshape(
        NUM_SUBCORES, NB2 * EDGE_BLOCK)
    zeros = jnp.zeros((N_PAD // NUM_SUBCORES, D // NUM_CORES), jnp.float32)

    def split_cols(y):
        y = jnp.pad(y, ((0, N_PAD - N), (0, 0)))
        return jnp.stack([y[:, :D // 2], y[:, D // 2:]])

    y0 = _mm(embedding, W0)
    p1 = _sc_aggregate(split_cols(y0), cols, rows, vals, zeros)
    h1, y1 = _norm_mm(p1, W1)
    p2 = _sc_aggregate(split_cols(y1), cols, rows, vals, zeros)
    return _final(embedding, h1, p2)


# speedup vs baseline: 3.1710x; 1.0271x over previous
import functools

import jax
import jax.numpy as jnp
from jax import lax
from jax.experimental import pallas as pl
from jax.experimental.pallas import tpu as pltpu
from jax.experimental.pallas import tpu_sc as plsc

N = 10000
D = 128
E = 320000

NUM_CORES = 2
NUM_SUBCORES = 16
LANES = 16
EDGE_BLOCK = 128
NB = 80
PE = NUM_CORES * NUM_SUBCORES * NB * EDGE_BLOCK
N_PAD = 10240
ROWS_PER_TILE = N_PAD // NUM_SUBCORES


def _make_sc_aggregate(n_pad, nb, edge_block, d=D, interpret=False):
  rows_per_tile = n_pad // NUM_SUBCORES
  dh = d // NUM_CORES
  chunk = 16
  nchunks = nb // chunk

  def body(y_hbm, cols_hbm, rows_hbm, vals_hbm, zeros_hbm, out_hbm,
           cols_a, cols_b, rows_a, rows_b, vals_a, vals_b, gath_a, gath_b,
           ystore, acc, sem_g0, sem_g1, sem_s0, sem_s1, sem_i):
    c = lax.axis_index("c")
    s = lax.axis_index("s")
    cols_ab = (cols_a, cols_b)
    rows_ab = (rows_a, rows_b)
    vals_ab = (vals_a, vals_b)

    rsl = pl.ds(s * rows_per_tile, rows_per_tile)
    pltpu.sync_copy(y_hbm.at[c, rsl], ystore.at[rsl])
    pltpu.sync_copy(zeros_hbm, acc.at[rsl])

    pltpu.sync_copy(cols_hbm.at[s, pl.ds(0, chunk)], cols_a)
    pltpu.sync_copy(rows_hbm.at[s, pl.ds(0, chunk)], rows_a)
    pltpu.sync_copy(vals_hbm.at[s, pl.ds(0, chunk * edge_block)], vals_a)

    plsc.subcore_barrier()

    dummy_g = y_hbm.at[0, pl.ds(0, edge_block)]
    dummy_cols = cols_hbm.at[0, pl.ds(0, chunk)]
    dummy_vals = vals_hbm.at[0, pl.ds(0, chunk * edge_block)]

    def drain(sem, dummy_src, dst):
        pltpu.make_async_copy(dummy_src, dst, sem).wait()

    def make_scale(vals_v, gath, k):
        base = k * edge_block

        @plsc.parallel_loop(0, edge_block, unroll=8)
        def _scale(e):
            fvec = jnp.full((LANES,), base + e, jnp.int32)
            val = plsc.load_gather(vals_v, [fvec])
            for j in range(dh // LANES):
                sl = pl.ds(j * LANES, LANES)
                gath[e, sl] = gath[e, sl] * val

    for g in range(nchunks):
        pc, pn = g % 2, (g + 1) % 2
        cols_c, rows_c, vals_c = cols_ab[pc], rows_ab[pc], vals_ab[pc]
        if g > 0:
            drain(sem_i, dummy_cols, cols_c)
            drain(sem_i, dummy_cols, rows_c)
            drain(sem_i, dummy_vals, vals_c)
        if g + 1 < nchunks:
            lo = (g + 1) * chunk
            pltpu.async_copy(cols_hbm.at[s, pl.ds(lo, chunk)],
                             cols_ab[pn], sem_i)
            pltpu.async_copy(rows_hbm.at[s, pl.ds(lo, chunk)],
                             rows_ab[pn], sem_i)
            pltpu.async_copy(vals_hbm.at[s, pl.ds(lo * edge_block,
                                                  chunk * edge_block)],
                             vals_ab[pn], sem_i)

        pltpu.async_copy(ystore.at[cols_c.at[0]], gath_a, sem_g0)
        pltpu.async_copy(ystore.at[cols_c.at[1]], gath_b, sem_g1)
        drain(sem_g0, dummy_g, gath_a)
        make_scale(vals_c, gath_a, 0)
        pltpu.async_copy(gath_a, acc.at[rows_c.at[0]], sem_s0, add=True)

        def pair(i, carry):
            k1 = 2 * i + 1
            drain(sem_s0, dummy_g, gath_a)
            pltpu.async_copy(ystore.at[cols_c.at[k1 + 1]], gath_a, sem_g0)
            drain(sem_g1, dummy_g, gath_b)
            make_scale(vals_c, gath_b, k1)
            pltpu.async_copy(gath_b, acc.at[rows_c.at[k1]], sem_s1, add=True)

            k2 = 2 * i + 2
            drain(sem_s1, dummy_g, gath_b)
            pltpu.async_copy(ystore.at[cols_c.at[k2 + 1]], gath_b, sem_g1)
            drain(sem_g0, dummy_g, gath_a)
            make_scale(vals_c, gath_a, k2)
            pltpu.async_copy(gath_a, acc.at[rows_c.at[k2]], sem_s0, add=True)
            return carry

        lax.fori_loop(0, (chunk - 2) // 2, pair, 0)

        drain(sem_s0, dummy_g, gath_a)
        drain(sem_g1, dummy_g, gath_b)
        make_scale(vals_c, gath_b, chunk - 1)
        pltpu.async_copy(gath_b, acc.at[rows_c.at[chunk - 1]], sem_s1,
                         add=True)
        drain(sem_s1, dummy_g, gath_b)

    plsc.subcore_barrier()
    pltpu.sync_copy(acc.at[rsl], out_hbm.at[c, rsl])

  return pl.kernel(
      body,
      out_type=jax.ShapeDtypeStruct((NUM_CORES, n_pad, d // NUM_CORES),
                                    jnp.float32),
      mesh=plsc.VectorSubcoreMesh(
          core_axis_name="c", subcore_axis_name="s",
          num_cores=NUM_CORES, num_subcores=NUM_SUBCORES),
      compiler_params=pltpu.CompilerParams(needs_layout_passes=False,
                                           use_tc_tiling_on_sc=False),
      interpret=interpret,
      scratch_types=[
          pltpu.VMEM((chunk, edge_block), jnp.int32),
          pltpu.VMEM((chunk, edge_block), jnp.int32),
          pltpu.VMEM((chunk, edge_block), jnp.int32),
          pltpu.VMEM((chunk, edge_block), jnp.int32),
          pltpu.VMEM((chunk * edge_block,), jnp.float32),
          pltpu.VMEM((chunk * edge_block,), jnp.float32),
          pltpu.VMEM((edge_block, D // NUM_CORES), jnp.float32),
          pltpu.VMEM((edge_block, D // NUM_CORES), jnp.float32),
          pltpu.VMEM_SHARED((n_pad, D // NUM_CORES), jnp.float32),
          pltpu.VMEM_SHARED((n_pad, D // NUM_CORES), jnp.float32),
          pltpu.SemaphoreType.DMA,
          pltpu.SemaphoreType.DMA,
          pltpu.SemaphoreType.DMA,
          pltpu.SemaphoreType.DMA,
          pltpu.SemaphoreType.DMA,
      ],
  )


NB2 = PE // (NUM_SUBCORES * EDGE_BLOCK)
_sc_aggregate = _make_sc_aggregate(N_PAD, NB2, EDGE_BLOCK)


def _split_write(y, y_ref):
    y_ref[0, :N] = y[:, :D // 2]
    y_ref[1, :N] = y[:, D // 2:]


def _mm_body(x_ref, w_ref, y_ref):
    _split_write(lax.dot_general(
        x_ref[...], w_ref[...], (((1,), (1,)), ((), ())),
        preferred_element_type=jnp.float32), y_ref)


_mm = pl.pallas_call(
    _mm_body,
    out_shape=jax.ShapeDtypeStruct((NUM_CORES, N_PAD, D // NUM_CORES),
                                   jnp.float32),
)


def _normalize(h):
    norm = jnp.sqrt(jnp.sum(h * h, axis=-1, keepdims=True))
    return h / jnp.maximum(norm, 1e-12)


def _agg_of(p_ref):
    return jnp.concatenate([p_ref[0, :N], p_ref[1, :N]], axis=-1)


def _norm_mm_body(p_ref, w_ref, h_ref, y_ref):
    agg = _agg_of(p_ref)
    h_ref[...] = _normalize(agg)
    _split_write(lax.dot_general(
        agg, w_ref[...], (((1,), (1,)), ((), ())),
        preferred_element_type=jnp.float32), y_ref)


_norm_mm = pl.pallas_call(
    _norm_mm_body,
    out_shape=(jax.ShapeDtypeStruct((N, D), jnp.float32),
               jax.ShapeDtypeStruct((NUM_CORES, N_PAD, D // NUM_CORES),
                                    jnp.float32)),
)


def _final_body(e_ref, h1_ref, p_ref, o_ref):
    h2 = _normalize(_agg_of(p_ref))
    o_ref[...] = (e_ref[...] + h1_ref[...] + h2) * (1.0 / 3.0)


_final = pl.pallas_call(
    _final_body,
    out_shape=jax.ShapeDtypeStruct((N, D), jnp.float32),
)


def kernel(embedding, adj_row, adj_col, adj_values, W0, W1):
    pad = PE - E
    cols = jnp.pad(adj_col.astype(jnp.int32), (0, pad)).reshape(
        NUM_SUBCORES, NB2, EDGE_BLOCK)
    rows = jnp.pad(adj_row.astype(jnp.int32), (0, pad)).reshape(
        NUM_SUBCORES, NB2, EDGE_BLOCK)
    vals = jnp.pad(adj_values, (0, pad)).reshape(
        NUM_SUBCORES, NB2 * EDGE_BLOCK)
    zeros = jnp.zeros((N_PAD // NUM_SUBCORES, D // NUM_CORES), jnp.float32)

    y0 = _mm(embedding, W0)
    p1 = _sc_aggregate(y0, cols, rows, vals, zeros)
    h1, y1 = _norm_mm(p1, W1)
    p2 = _sc_aggregate(y1, cols, rows, vals, zeros)
    return _final(embedding, h1, p2)
